# baseline (speedup 1.0000x reference)
import functools

import jax
import jax.numpy as jnp
from jax import lax
from jax.experimental import pallas as pl
from jax.experimental.pallas import tpu as pltpu
from jax.experimental.pallas import tpu_sc as plsc

NODE_D = 128
HID = 64
EDGE_D = 16

_NC = 2
_NS = 16
_NW = _NC * _NS


def _elu(x):
    return jnp.where(x > 0, x, jnp.exp(x) - 1.0)


def _softplus(x):
    return jnp.maximum(x, 0.0) + jnp.log(1.0 + jnp.exp(-jnp.abs(x)))


def _precompute_body(nf_ref, wst_ref, wrt_ref, ps_ref, pr_ref):
    nf = nf_ref[...]
    ps_ref[...] = jnp.dot(nf, wst_ref[...], preferred_element_type=jnp.float32)
    pr_ref[...] = jnp.dot(nf, wrt_ref[...], preferred_element_type=jnp.float32)


def _mlp_body(g2_ref, eftlo_ref, efthi_ref, w1et_ref, b1_ref, w2t_ref, b2_ref,
              w3_ref, b3_ref, outlo_ref, outhi_ref):
    g2 = g2_ref[...]
    x = jnp.concatenate([g2[:, :HID], g2[:, HID:]], axis=0)
    eft = jnp.concatenate([eftlo_ref[...], efthi_ref[...]], axis=1)
    a = lax.dot_general(eft, w1et_ref[...], (((0,), (0,)), ((), ())),
                        preferred_element_type=jnp.float32)
    h1 = _elu(x + a + b1_ref[...])
    h2 = _elu(jnp.dot(h1, w2t_ref[...], preferred_element_type=jnp.float32)
              + b2_ref[...])
    z = lax.dot_general(w3_ref[...], h2, (((1,), (1,)), ((), ())),
                        preferred_element_type=jnp.float32)
    sp = _softplus(z + b3_ref[...])
    half = sp.shape[1] // 2
    outlo_ref[0] = sp[:, :half]
    outhi_ref[0] = sp[:, half:]


def _make_gather(n_edges, cpairs):
    half = n_edges // 2
    ppw = half // _NW
    nchunk = ppw // cpairs
    mesh = plsc.VectorSubcoreMesh(core_axis_name="c", subcore_axis_name="s")

    @functools.partial(
        pl.kernel,
        mesh=mesh,
        compiler_params=pltpu.CompilerParams(use_tc_tiling_on_sc=False),
        out_type=jax.ShapeDtypeStruct((half, 2 * HID), jnp.float32),
        scratch_types=[
            pltpu.VMEM((2 * ppw,), jnp.int32),
            pltpu.VMEM((2 * ppw,), jnp.int32),
            pltpu.VMEM((cpairs, HID), jnp.float32),
            pltpu.VMEM((cpairs, HID), jnp.float32),
            pltpu.VMEM((cpairs, HID), jnp.float32),
            pltpu.VMEM((cpairs, HID), jnp.float32),
            pltpu.VMEM((cpairs, HID), jnp.float32),
            pltpu.VMEM((cpairs, HID), jnp.float32),
            pltpu.SemaphoreType.DMA,
            pltpu.SemaphoreType.DMA,
            pltpu.SemaphoreType.DMA,
            pltpu.SemaphoreType.DMA,
        ],
    )
    def _gather(ps_hbm, pr_hbm, s_hbm, r_hbm, out_hbm, idx_s, idx_r,
                bs0, br0, bw0, bs1, br1, bw1, semg0, semg1, semw0, semw1):
        wid = lax.axis_index("s") * _NC + lax.axis_index("c")
        base = wid * ppw
        bufs = ((bs0, br0, bw0, semg0, semw0), (bs1, br1, bw1, semg1, semw1))

        pltpu.sync_copy(s_hbm.at[pl.ds(base, ppw)], idx_s.at[pl.ds(0, ppw)])
        pltpu.sync_copy(s_hbm.at[pl.ds(half + base, ppw)],
                        idx_s.at[pl.ds(ppw, ppw)])
        pltpu.sync_copy(r_hbm.at[pl.ds(base, ppw)], idx_r.at[pl.ds(0, ppw)])
        pltpu.sync_copy(r_hbm.at[pl.ds(half + base, ppw)],
                        idx_r.at[pl.ds(ppw, ppw)])

        def issue_gathers(b, k):
            bs, br, _, semg, _ = bufs[b]
            ioff = b * ppw + k * cpairs
            cs = pltpu.async_copy(
                ps_hbm.at[idx_s.at[pl.ds(ioff, cpairs)]], bs, semg)
            cr = pltpu.async_copy(
                pr_hbm.at[idx_r.at[pl.ds(ioff, cpairs)]], br, semg)
            return cs, cr

        issue_gathers(0, 0)
        issue_gathers(1, 0)

        def chunk_body(k, carry):
            for b in (0, 1):
                bs, br, bw, semg, semw = bufs[b]
                coff = b * HID
                row = base + k * cpairs

                pltpu.make_async_copy(
                    ps_hbm.at[idx_s.at[pl.ds(0, cpairs)]], bs, semg).wait()
                pltpu.make_async_copy(
                    pr_hbm.at[idx_r.at[pl.ds(0, cpairs)]], br, semg).wait()

                @pl.when(k >= 1)
                def _():
                    pltpu.make_async_copy(
                        bw, out_hbm.at[pl.ds(row - cpairs, cpairs),
                                       pl.ds(coff, HID)], semw).wait()

                def add_row(rr, inner):
                    for c4 in range(HID // 16):
                        sl = pl.ds(c4 * 16, 16)
                        bw[rr, sl] = bs[rr, sl] + br[rr, sl]
                    return inner

                lax.fori_loop(0, cpairs, add_row, 0)

                @pl.when(k < nchunk - 1)
                def _():
                    issue_gathers(b, k + 1)

                pltpu.async_copy(
                    bw, out_hbm.at[pl.ds(row, cpairs), pl.ds(coff, HID)], semw)
            return carry

        lax.fori_loop(0, nchunk, chunk_body, 0)

        for b in (0, 1):
            _, _, bw, _, semw = bufs[b]
            row = base + (nchunk - 1) * cpairs
            pltpu.make_async_copy(
                bw, out_hbm.at[pl.ds(row, cpairs), pl.ds(b * HID, HID)],
                semw).wait()

    return _gather


def kernel(node_features, edge_index, edge_features, W1, b1, W2, b2, W3, b3):
    n_nodes = node_features.shape[0]
    n_edges = edge_features.shape[0]
    half = n_edges // 2

    s32 = edge_index[0].astype(jnp.int32)
    r32 = edge_index[1].astype(jnp.int32)
    eft = edge_features.T
    w1st = W1[:, :NODE_D].T
    w1rt = W1[:, NODE_D:2 * NODE_D].T
    w1et = W1[:, 2 * NODE_D:].T
    b1_2 = b1.reshape(1, HID)
    b2_2 = b2.reshape(1, HID)
    b3_2 = b3.reshape(1, 1)

    ps, pr = pl.pallas_call(
        _precompute_body,
        out_shape=(
            jax.ShapeDtypeStruct((n_nodes, HID), jnp.float32),
            jax.ShapeDtypeStruct((n_nodes, HID), jnp.float32),
        ),
    )(node_features, w1st, w1rt)

    slices = ((0, 128000, 640), (128000, 320000, 640))
    pieces = []
    for start, end, rows in slices:
        e_s = end - start
        half_s = e_s // 2
        g2 = _make_gather(e_s, 200)(ps, pr, s32[start:end], r32[start:end])

        nblocks = half_s // rows
        lo_base = start // rows
        hi_base = (start + half_s) // rows
        out_lo, out_hi = pl.pallas_call(
            _mlp_body,
            grid=(nblocks,),
            in_specs=[
                pl.BlockSpec((rows, 2 * HID), lambda i: (i, 0)),
                pl.BlockSpec((EDGE_D, rows), lambda i, _b=lo_base: (0, i + _b)),
                pl.BlockSpec((EDGE_D, rows), lambda i, _b=hi_base: (0, i + _b)),
                pl.BlockSpec((EDGE_D, HID), lambda i: (0, 0)),
                pl.BlockSpec((1, HID), lambda i: (0, 0)),
                pl.BlockSpec((HID, HID), lambda i: (0, 0)),
                pl.BlockSpec((1, HID), lambda i: (0, 0)),
                pl.BlockSpec((1, HID), lambda i: (0, 0)),
                pl.BlockSpec((1, 1), lambda i: (0, 0)),
            ],
            out_specs=(
                pl.BlockSpec((1, 1, rows), lambda i: (i, 0, 0)),
                pl.BlockSpec((1, 1, rows), lambda i: (i, 0, 0)),
            ),
            out_shape=(
                jax.ShapeDtypeStruct((nblocks, 1, rows), jnp.float32),
                jax.ShapeDtypeStruct((nblocks, 1, rows), jnp.float32),
            ),
        )(g2, eft, eft, w1et, b1_2, W2.T, b2_2, W3, b3_2)
        pieces.append(out_lo.reshape(half_s))
        pieces.append(out_hi.reshape(half_s))
    return jnp.concatenate(pieces)


# speedup vs baseline: 1.4728x; 1.4728x over previous
import functools

import jax
import jax.numpy as jnp
from jax import lax
from jax.experimental import pallas as pl
from jax.experimental.pallas import tpu as pltpu
from jax.experimental.pallas import tpu_sc as plsc

NODE_D = 128
HID = 64
EDGE_D = 16

_NC = 2
_NS = 16
_NW = _NC * _NS


def _elu(x):
    return jnp.where(x > 0, x, jnp.exp(x) - 1.0)


def _softplus(x):
    return jnp.maximum(x, 0.0) + jnp.log(1.0 + jnp.exp(-jnp.abs(x)))


def _precompute_body(nf_ref, wst_ref, wrt_ref, ps_ref, pr_ref):
    nf = nf_ref[...]
    ps_ref[...] = jnp.dot(nf, wst_ref[...], preferred_element_type=jnp.float32)
    pr_ref[...] = jnp.dot(nf, wrt_ref[...], preferred_element_type=jnp.float32)


def _mlp_body(g2_ref, eftlo_ref, efthi_ref, w1et_ref, b1_ref, w2t_ref, b2_ref,
              w3_ref, b3_ref, outlo_ref, outhi_ref):
    g2 = g2_ref[...]
    x = jnp.concatenate([g2[:, :HID], g2[:, HID:]], axis=0)
    eft = jnp.concatenate([eftlo_ref[...], efthi_ref[...]], axis=1)
    a = lax.dot_general(eft, w1et_ref[...], (((0,), (0,)), ((), ())),
                        preferred_element_type=jnp.float32)
    h1 = _elu(x + a + b1_ref[...])
    h2 = _elu(jnp.dot(h1, w2t_ref[...], preferred_element_type=jnp.float32)
              + b2_ref[...])
    z = lax.dot_general(w3_ref[...], h2, (((1,), (1,)), ((), ())),
                        preferred_element_type=jnp.float32)
    sp = _softplus(z + b3_ref[...])
    half = sp.shape[1] // 2
    outlo_ref[0] = sp[:, :half]
    outhi_ref[0] = sp[:, half:]


def _make_gather(n_edges, cpairs):
    half = n_edges // 2
    ppw = half // _NW
    nchunk = ppw // cpairs
    mesh = plsc.VectorSubcoreMesh(core_axis_name="c", subcore_axis_name="s")

    @functools.partial(
        pl.kernel,
        mesh=mesh,
        compiler_params=pltpu.CompilerParams(use_tc_tiling_on_sc=False),
        out_type=jax.ShapeDtypeStruct((half, 2 * HID), jnp.float32),
        scratch_types=[
            pltpu.VMEM((2 * ppw,), jnp.int32),
            pltpu.VMEM((2 * ppw,), jnp.int32),
            pltpu.VMEM((cpairs, HID), jnp.float32),
            pltpu.VMEM((cpairs, HID), jnp.float32),
            pltpu.VMEM((cpairs, HID), jnp.float32),
            pltpu.VMEM((cpairs, HID), jnp.float32),
            pltpu.VMEM((cpairs, HID), jnp.float32),
            pltpu.VMEM((cpairs, HID), jnp.float32),
            pltpu.SemaphoreType.DMA,
            pltpu.SemaphoreType.DMA,
            pltpu.SemaphoreType.DMA,
            pltpu.SemaphoreType.DMA,
        ],
    )
    def _gather(ps_hbm, pr_hbm, s_hbm, r_hbm, out_hbm, idx_s, idx_r,
                bs0, br0, bw0, bs1, br1, bw1, semg0, semg1, semw0, semw1):
        wid = lax.axis_index("s") * _NC + lax.axis_index("c")
        base = wid * ppw
        bufs = ((bs0, br0, bw0, semg0, semw0), (bs1, br1, bw1, semg1, semw1))

        pltpu.sync_copy(s_hbm.at[pl.ds(base, ppw)], idx_s.at[pl.ds(0, ppw)])
        pltpu.sync_copy(s_hbm.at[pl.ds(half + base, ppw)],
                        idx_s.at[pl.ds(ppw, ppw)])
        pltpu.sync_copy(r_hbm.at[pl.ds(base, ppw)], idx_r.at[pl.ds(0, ppw)])
        pltpu.sync_copy(r_hbm.at[pl.ds(half + base, ppw)],
                        idx_r.at[pl.ds(ppw, ppw)])

        def issue_gathers(b, k):
            bs, br, _, semg, _ = bufs[b]
            ioff = b * ppw + k * cpairs
            cs = pltpu.async_copy(
                ps_hbm.at[idx_s.at[pl.ds(ioff, cpairs)]], bs, semg)
            cr = pltpu.async_copy(
                pr_hbm.at[idx_r.at[pl.ds(ioff, cpairs)]], br, semg)
            return cs, cr

        issue_gathers(0, 0)
        issue_gathers(1, 0)

        def chunk_body(k, carry):
            for b in (0, 1):
                bs, br, bw, semg, semw = bufs[b]
                coff = b * HID
                row = base + k * cpairs

                pltpu.make_async_copy(
                    ps_hbm.at[idx_s.at[pl.ds(0, cpairs)]], bs, semg).wait()
                pltpu.make_async_copy(
                    pr_hbm.at[idx_r.at[pl.ds(0, cpairs)]], br, semg).wait()

                @pl.when(k >= 1)
                def _():
                    pltpu.make_async_copy(
                        bw, out_hbm.at[pl.ds(row - cpairs, cpairs),
                                       pl.ds(coff, HID)], semw).wait()

                def add_row(rr, inner):
                    for c4 in range(HID // 16):
                        sl = pl.ds(c4 * 16, 16)
                        bw[rr, sl] = bs[rr, sl] + br[rr, sl]
                    return inner

                lax.fori_loop(0, cpairs, add_row, 0)

                @pl.when(k < nchunk - 1)
                def _():
                    issue_gathers(b, k + 1)

                pltpu.async_copy(
                    bw, out_hbm.at[pl.ds(row, cpairs), pl.ds(coff, HID)], semw)
            return carry

        lax.fori_loop(0, nchunk, chunk_body, 0)

        for b in (0, 1):
            _, _, bw, _, semw = bufs[b]
            row = base + (nchunk - 1) * cpairs
            pltpu.make_async_copy(
                bw, out_hbm.at[pl.ds(row, cpairs), pl.ds(b * HID, HID)],
                semw).wait()

    return _gather


def kernel(node_features, edge_index, edge_features, W1, b1, W2, b2, W3, b3):
    n_nodes = node_features.shape[0]
    n_edges = edge_features.shape[0]
    half = n_edges // 2

    s32 = edge_index[0].astype(jnp.int32)
    r32 = edge_index[1].astype(jnp.int32)
    eft = edge_features.T
    w1st = W1[:, :NODE_D].T
    w1rt = W1[:, NODE_D:2 * NODE_D].T
    w1et = W1[:, 2 * NODE_D:].T
    b1_2 = b1.reshape(1, HID)
    b2_2 = b2.reshape(1, HID)
    b3_2 = b3.reshape(1, 1)

    ps, pr = pl.pallas_call(
        _precompute_body,
        out_shape=(
            jax.ShapeDtypeStruct((n_nodes, HID), jnp.float32),
            jax.ShapeDtypeStruct((n_nodes, HID), jnp.float32),
        ),
    )(node_features, w1st, w1rt)

    slices = ((0, 128000, 3200), (128000, 320000, 3200))
    pieces = []
    for start, end, rows in slices:
        e_s = end - start
        half_s = e_s // 2
        g2 = _make_gather(e_s, 200)(ps, pr, s32[start:end], r32[start:end])

        nblocks = half_s // rows
        lo_base = start // rows
        hi_base = (start + half_s) // rows
        out_lo, out_hi = pl.pallas_call(
            _mlp_body,
            grid=(nblocks,),
            in_specs=[
                pl.BlockSpec((rows, 2 * HID), lambda i: (i, 0)),
                pl.BlockSpec((EDGE_D, rows), lambda i, _b=lo_base: (0, i + _b)),
                pl.BlockSpec((EDGE_D, rows), lambda i, _b=hi_base: (0, i + _b)),
                pl.BlockSpec((EDGE_D, HID), lambda i: (0, 0)),
                pl.BlockSpec((1, HID), lambda i: (0, 0)),
                pl.BlockSpec((HID, HID), lambda i: (0, 0)),
                pl.BlockSpec((1, HID), lambda i: (0, 0)),
                pl.BlockSpec((1, HID), lambda i: (0, 0)),
                pl.BlockSpec((1, 1), lambda i: (0, 0)),
            ],
            out_specs=(
                pl.BlockSpec((1, 1, rows), lambda i: (i, 0, 0)),
                pl.BlockSpec((1, 1, rows), lambda i: (i, 0, 0)),
            ),
            out_shape=(
                jax.ShapeDtypeStruct((nblocks, 1, rows), jnp.float32),
                jax.ShapeDtypeStruct((nblocks, 1, rows), jnp.float32),
            ),
        )(g2, eft, eft, w1et, b1_2, W2.T, b2_2, W3, b3_2)
        pieces.append(out_lo.reshape(half_s))
        pieces.append(out_hi.reshape(half_s))
    return jnp.concatenate(pieces)


# speedup vs baseline: 1.5978x; 1.0849x over previous
import functools

import jax
import jax.numpy as jnp
from jax import lax
from jax.experimental import pallas as pl
from jax.experimental.pallas import tpu as pltpu
from jax.experimental.pallas import tpu_sc as plsc

NODE_D = 128
HID = 64
EDGE_D = 16

_NC = 2
_NS = 16
_NW = _NC * _NS


def _elu(x):
    return jnp.where(x > 0, x, jnp.exp(x) - 1.0)


def _softplus(x):
    return jnp.maximum(x, 0.0) + jnp.log(1.0 + jnp.exp(-jnp.abs(x)))


def _precompute_body(nf2_ref, wst_ref, wrt_ref, ei_ref, ps_ref, pr_ref,
                     sa_ref, ra_ref, sb_ref, rb_ref):
    nf2 = nf2_ref[...]
    ps_ref[...] = jnp.dot(nf2, wst_ref[...], preferred_element_type=jnp.float32)
    pr_ref[...] = jnp.dot(nf2, wrt_ref[...], preferred_element_type=jnp.float32)
    cut = sa_ref.shape[0]
    s_row = ei_ref[0]
    r_row = ei_ref[1]
    sa_ref[...] = s_row[:cut]
    sb_ref[...] = s_row[cut:]
    ra_ref[...] = r_row[:cut]
    rb_ref[...] = r_row[cut:]


def _mlp_body(g2_ref, eftlo_ref, efthi_ref, w1et_ref, b1_ref, w2t_ref, b2_ref,
              w3_ref, b3_ref, outlo_ref, outhi_ref):
    g2 = g2_ref[...]
    x = jnp.concatenate([g2[:, :HID], g2[:, HID:]], axis=0)
    eft = jnp.concatenate([eftlo_ref[...], efthi_ref[...]], axis=1)
    a = lax.dot_general(eft, w1et_ref[...], (((0,), (0,)), ((), ())),
                        preferred_element_type=jnp.float32)
    h1 = _elu(x + a + b1_ref[...])
    h2 = _elu(jnp.dot(h1, w2t_ref[...], preferred_element_type=jnp.float32)
              + b2_ref[...])
    z = lax.dot_general(w3_ref[...], h2, (((1,), (1,)), ((), ())),
                        preferred_element_type=jnp.float32)
    sp = _softplus(z + b3_ref[...])
    half = sp.shape[1] // 2
    outlo_ref[0] = sp[:, :half]
    outhi_ref[0] = sp[:, half:]


def _make_gather(n_edges, cpairs):
    half = n_edges // 2
    ppw = half // _NW
    nchunk = ppw // cpairs
    mesh = plsc.VectorSubcoreMesh(core_axis_name="c", subcore_axis_name="s")

    @functools.partial(
        pl.kernel,
        mesh=mesh,
        compiler_params=pltpu.CompilerParams(use_tc_tiling_on_sc=False),
        out_type=jax.ShapeDtypeStruct((half, 2 * HID), jnp.float32),
        scratch_types=[
            pltpu.VMEM((2 * ppw,), jnp.int32),
            pltpu.VMEM((2 * ppw,), jnp.int32),
            pltpu.VMEM((cpairs, HID), jnp.float32),
            pltpu.VMEM((cpairs, HID), jnp.float32),
            pltpu.VMEM((cpairs, HID), jnp.float32),
            pltpu.VMEM((cpairs, HID), jnp.float32),
            pltpu.VMEM((cpairs, HID), jnp.float32),
            pltpu.VMEM((cpairs, HID), jnp.float32),
            pltpu.SemaphoreType.DMA,
            pltpu.SemaphoreType.DMA,
            pltpu.SemaphoreType.DMA,
            pltpu.SemaphoreType.DMA,
        ],
    )
    def _gather(ps_hbm, pr_hbm, s_hbm, r_hbm, out_hbm, idx_s, idx_r,
                bs0, br0, bw0, bs1, br1, bw1, semg0, semg1, semw0, semw1):
        wid = lax.axis_index("s") * _NC + lax.axis_index("c")
        base = wid * ppw
        bufs = ((bs0, br0, bw0, semg0, semw0), (bs1, br1, bw1, semg1, semw1))

        pltpu.sync_copy(s_hbm.at[pl.ds(base, ppw)], idx_s.at[pl.ds(0, ppw)])
        pltpu.sync_copy(s_hbm.at[pl.ds(half + base, ppw)],
                        idx_s.at[pl.ds(ppw, ppw)])
        pltpu.sync_copy(r_hbm.at[pl.ds(base, ppw)], idx_r.at[pl.ds(0, ppw)])
        pltpu.sync_copy(r_hbm.at[pl.ds(half + base, ppw)],
                        idx_r.at[pl.ds(ppw, ppw)])

        def issue_gathers(b, k):
            bs, br, _, semg, _ = bufs[b]
            ioff = b * ppw + k * cpairs
            cs = pltpu.async_copy(
                ps_hbm.at[idx_s.at[pl.ds(ioff, cpairs)]], bs, semg)
            cr = pltpu.async_copy(
                pr_hbm.at[idx_r.at[pl.ds(ioff, cpairs)]], br, semg)
            return cs, cr

        issue_gathers(0, 0)
        issue_gathers(1, 0)

        def chunk_body(k, carry):
            for b in (0, 1):
                bs, br, bw, semg, semw = bufs[b]
                coff = b * HID
                row = base + k * cpairs

                pltpu.make_async_copy(
                    ps_hbm.at[idx_s.at[pl.ds(0, cpairs)]], bs, semg).wait()
                pltpu.make_async_copy(
                    pr_hbm.at[idx_r.at[pl.ds(0, cpairs)]], br, semg).wait()

                @pl.when(k >= 1)
                def _():
                    pltpu.make_async_copy(
                        bw, out_hbm.at[pl.ds(row - cpairs, cpairs),
                                       pl.ds(coff, HID)], semw).wait()

                def add_row(rr, inner):
                    for c4 in range(HID // 16):
                        sl = pl.ds(c4 * 16, 16)
                        bw[rr, sl] = bs[rr, sl] + br[rr, sl]
                    return inner

                lax.fori_loop(0, cpairs, add_row, 0)

                @pl.when(k < nchunk - 1)
                def _():
                    issue_gathers(b, k + 1)

                pltpu.async_copy(
                    bw, out_hbm.at[pl.ds(row, cpairs), pl.ds(coff, HID)], semw)
            return carry

        lax.fori_loop(0, nchunk, chunk_body, 0)

        for b in (0, 1):
            _, _, bw, _, semw = bufs[b]
            row = base + (nchunk - 1) * cpairs
            pltpu.make_async_copy(
                bw, out_hbm.at[pl.ds(row, cpairs), pl.ds(b * HID, HID)],
                semw).wait()

    return _gather


def kernel(node_features, edge_index, edge_features, W1, b1, W2, b2, W3, b3):
    n_nodes = node_features.shape[0]
    n_edges = edge_features.shape[0]
    half = n_edges // 2

    eft = edge_features.T
    w1st = W1[:, :NODE_D].T
    w1rt = W1[:, NODE_D:2 * NODE_D].T
    w1et = W1[:, 2 * NODE_D:].T
    b1_2 = b1.reshape(1, HID)
    b2_2 = b2.reshape(1, HID)
    b3_2 = b3.reshape(1, 1)

    cut = 128000
    ei32 = jnp.asarray(edge_index, jnp.int32)
    nf2 = node_features.reshape(n_nodes // 2, 2 * NODE_D)
    zer = jnp.zeros((NODE_D, HID), jnp.float32)
    wst_bd = jnp.block([[w1st, zer], [zer, w1st]])
    wrt_bd = jnp.block([[w1rt, zer], [zer, w1rt]])

    ps_p, pr_p, s_a, r_a, s_b, r_b = pl.pallas_call(
        _precompute_body,
        out_shape=(
            jax.ShapeDtypeStruct((n_nodes // 2, 2 * HID), jnp.float32),
            jax.ShapeDtypeStruct((n_nodes // 2, 2 * HID), jnp.float32),
            jax.ShapeDtypeStruct((cut,), jnp.int32),
            jax.ShapeDtypeStruct((cut,), jnp.int32),
            jax.ShapeDtypeStruct((n_edges - cut,), jnp.int32),
            jax.ShapeDtypeStruct((n_edges - cut,), jnp.int32),
        ),
    )(nf2, wst_bd, wrt_bd, ei32)
    ps = ps_p.reshape(n_nodes, HID)
    pr = pr_p.reshape(n_nodes, HID)

    slices = ((0, cut, 6400, s_a, r_a), (cut, n_edges, 6400, s_b, r_b))
    pieces = []
    for start, end, rows, s_sl, r_sl in slices:
        e_s = end - start
        half_s = e_s // 2
        g2 = _make_gather(e_s, 200)(ps, pr, s_sl, r_sl)

        nblocks = half_s // rows
        lo_base = start // rows
        hi_base = (start + half_s) // rows
        out_lo, out_hi = pl.pallas_call(
            _mlp_body,
            grid=(nblocks,),
            in_specs=[
                pl.BlockSpec((rows, 2 * HID), lambda i: (i, 0)),
                pl.BlockSpec((EDGE_D, rows), lambda i, _b=lo_base: (0, i + _b)),
                pl.BlockSpec((EDGE_D, rows), lambda i, _b=hi_base: (0, i + _b)),
                pl.BlockSpec((EDGE_D, HID), lambda i: (0, 0)),
                pl.BlockSpec((1, HID), lambda i: (0, 0)),
                pl.BlockSpec((HID, HID), lambda i: (0, 0)),
                pl.BlockSpec((1, HID), lambda i: (0, 0)),
                pl.BlockSpec((1, HID), lambda i: (0, 0)),
                pl.BlockSpec((1, 1), lambda i: (0, 0)),
            ],
            out_specs=(
                pl.BlockSpec((1, 1, rows), lambda i: (i, 0, 0)),
                pl.BlockSpec((1, 1, rows), lambda i: (i, 0, 0)),
            ),
            out_shape=(
                jax.ShapeDtypeStruct((nblocks, 1, rows), jnp.float32),
                jax.ShapeDtypeStruct((nblocks, 1, rows), jnp.float32),
            ),
        )(g2, eft, eft, w1et, b1_2, W2.T, b2_2, W3, b3_2)
        pieces.append(out_lo.reshape(half_s))
        pieces.append(out_hi.reshape(half_s))
    return jnp.concatenate(pieces)


# speedup vs baseline: 1.6585x; 1.0380x over previous
import functools

import jax
import jax.numpy as jnp
from jax import lax
from jax.experimental import pallas as pl
from jax.experimental.pallas import tpu as pltpu
from jax.experimental.pallas import tpu_sc as plsc

NODE_D = 128
HID = 64
EDGE_D = 16

_NC = 2
_NS = 16
_NW = _NC * _NS


def _elu(x):
    return jnp.where(x > 0, x, jnp.exp(x) - 1.0)


def _softplus(x):
    return jnp.maximum(x, 0.0) + jnp.log(1.0 + jnp.exp(-jnp.abs(x)))


def _precompute_body(nf_ref, wst_ref, wrt_ref, ei_ref, ps_ref, pr_ref,
                     sa_ref, ra_ref, sb_ref, rb_ref):
    nf = nf_ref[...]
    hn = nf.shape[0] // 2
    wst = wst_ref[...]
    wrt = wrt_ref[...]
    ps_ref[...] = jnp.concatenate(
        [jnp.dot(nf[:hn], wst, preferred_element_type=jnp.float32),
         jnp.dot(nf[hn:], wst, preferred_element_type=jnp.float32)], axis=1)
    pr_ref[...] = jnp.concatenate(
        [jnp.dot(nf[:hn], wrt, preferred_element_type=jnp.float32),
         jnp.dot(nf[hn:], wrt, preferred_element_type=jnp.float32)], axis=1)
    cut = sa_ref.shape[0]
    s_row = ei_ref[0]
    r_row = ei_ref[1]
    s_row = 2 * s_row - jnp.where(s_row < hn, 0, 2 * hn - 1)
    r_row = 2 * r_row - jnp.where(r_row < hn, 0, 2 * hn - 1)
    sa_ref[...] = s_row[:cut]
    sb_ref[...] = s_row[cut:]
    ra_ref[...] = r_row[:cut]
    rb_ref[...] = r_row[cut:]


def _mlp_body(g2a_ref, g2b_ref, eftlo_ref, efthi_ref, w1et_ref, b1_ref,
              w2t_ref, b2_ref, w3_ref, b3_ref, outlo_ref, outhi_ref):
    g2 = jnp.concatenate([g2a_ref[...], g2b_ref[...]], axis=0)
    x = jnp.concatenate([g2[:, :HID], g2[:, HID:]], axis=0)
    eft = jnp.concatenate([eftlo_ref[...], efthi_ref[...]], axis=1)
    a = lax.dot_general(eft, w1et_ref[...], (((0,), (0,)), ((), ())),
                        preferred_element_type=jnp.float32)
    h1 = _elu(x + a + b1_ref[...])
    h2 = _elu(jnp.dot(h1, w2t_ref[...], preferred_element_type=jnp.float32)
              + b2_ref[...])
    z = lax.dot_general(w3_ref[...], h2, (((1,), (1,)), ((), ())),
                        preferred_element_type=jnp.float32)
    sp = _softplus(z + b3_ref[...])
    half = sp.shape[1] // 2
    outlo_ref[0] = sp[:, :half]
    outhi_ref[0] = sp[:, half:]


def _make_gather(n_edges, cpairs):
    half = n_edges // 2
    ppw = half // _NW
    nchunk = ppw // cpairs
    mesh = plsc.VectorSubcoreMesh(core_axis_name="c", subcore_axis_name="s")

    @functools.partial(
        pl.kernel,
        mesh=mesh,
        compiler_params=pltpu.CompilerParams(use_tc_tiling_on_sc=False),
        out_type=jax.ShapeDtypeStruct((half, 2 * HID), jnp.float32),
        scratch_types=[
            pltpu.VMEM((2 * ppw,), jnp.int32),
            pltpu.VMEM((2 * ppw,), jnp.int32),
            pltpu.VMEM((cpairs, HID), jnp.float32),
            pltpu.VMEM((cpairs, HID), jnp.float32),
            pltpu.VMEM((cpairs, HID), jnp.float32),
            pltpu.VMEM((cpairs, HID), jnp.float32),
            pltpu.VMEM((cpairs, HID), jnp.float32),
            pltpu.VMEM((cpairs, HID), jnp.float32),
            pltpu.SemaphoreType.DMA,
            pltpu.SemaphoreType.DMA,
            pltpu.SemaphoreType.DMA,
            pltpu.SemaphoreType.DMA,
        ],
    )
    def _gather(ps_hbm, pr_hbm, s_hbm, r_hbm, out_hbm, idx_s, idx_r,
                bs0, br0, bw0, bs1, br1, bw1, semg0, semg1, semw0, semw1):
        wid = lax.axis_index("s") * _NC + lax.axis_index("c")
        base = wid * ppw
        bufs = ((bs0, br0, bw0, semg0, semw0), (bs1, br1, bw1, semg1, semw1))

        pltpu.sync_copy(s_hbm.at[pl.ds(base, ppw)], idx_s.at[pl.ds(0, ppw)])
        pltpu.sync_copy(s_hbm.at[pl.ds(half + base, ppw)],
                        idx_s.at[pl.ds(ppw, ppw)])
        pltpu.sync_copy(r_hbm.at[pl.ds(base, ppw)], idx_r.at[pl.ds(0, ppw)])
        pltpu.sync_copy(r_hbm.at[pl.ds(half + base, ppw)],
                        idx_r.at[pl.ds(ppw, ppw)])

        def issue_gathers(b, k):
            bs, br, _, semg, _ = bufs[b]
            ioff = b * ppw + k * cpairs
            cs = pltpu.async_copy(
                ps_hbm.at[idx_s.at[pl.ds(ioff, cpairs)]], bs, semg)
            cr = pltpu.async_copy(
                pr_hbm.at[idx_r.at[pl.ds(ioff, cpairs)]], br, semg)
            return cs, cr

        issue_gathers(0, 0)
        issue_gathers(1, 0)

        def chunk_body(k, carry):
            for b in (0, 1):
                bs, br, bw, semg, semw = bufs[b]
                coff = b * HID
                row = base + k * cpairs

                pltpu.make_async_copy(
                    ps_hbm.at[idx_s.at[pl.ds(0, cpairs)]], bs, semg).wait()
                pltpu.make_async_copy(
                    pr_hbm.at[idx_r.at[pl.ds(0, cpairs)]], br, semg).wait()

                @pl.when(k >= 1)
                def _():
                    pltpu.make_async_copy(
                        bw, out_hbm.at[pl.ds(row - cpairs, cpairs),
                                       pl.ds(coff, HID)], semw).wait()

                def add_row(rr, inner):
                    for c4 in range(HID // 16):
                        sl = pl.ds(c4 * 16, 16)
                        bw[rr, sl] = bs[rr, sl] + br[rr, sl]
                    return inner

                lax.fori_loop(0, cpairs, add_row, 0)

                @pl.when(k < nchunk - 1)
                def _():
                    issue_gathers(b, k + 1)

                pltpu.async_copy(
                    bw, out_hbm.at[pl.ds(row, cpairs), pl.ds(coff, HID)], semw)
            return carry

        lax.fori_loop(0, nchunk, chunk_body, 0)

        for b in (0, 1):
            _, _, bw, _, semw = bufs[b]
            row = base + (nchunk - 1) * cpairs
            pltpu.make_async_copy(
                bw, out_hbm.at[pl.ds(row, cpairs), pl.ds(b * HID, HID)],
                semw).wait()

    return _gather


def kernel(node_features, edge_index, edge_features, W1, b1, W2, b2, W3, b3):
    n_nodes = node_features.shape[0]
    n_edges = edge_features.shape[0]
    half = n_edges // 2

    eft = edge_features.T
    w1st = W1[:, :NODE_D].T
    w1rt = W1[:, NODE_D:2 * NODE_D].T
    w1et = W1[:, 2 * NODE_D:].T
    b1_2 = b1.reshape(1, HID)
    b2_2 = b2.reshape(1, HID)
    b3_2 = b3.reshape(1, 1)

    cut = 128000
    ei32 = jnp.asarray(edge_index, jnp.int32)

    ps_p, pr_p, s_a, r_a, s_b, r_b = pl.pallas_call(
        _precompute_body,
        out_shape=(
            jax.ShapeDtypeStruct((n_nodes // 2, 2 * HID), jnp.float32),
            jax.ShapeDtypeStruct((n_nodes // 2, 2 * HID), jnp.float32),
            jax.ShapeDtypeStruct((cut,), jnp.int32),
            jax.ShapeDtypeStruct((cut,), jnp.int32),
            jax.ShapeDtypeStruct((n_edges - cut,), jnp.int32),
            jax.ShapeDtypeStruct((n_edges - cut,), jnp.int32),
        ),
    )(node_features, w1st, w1rt, ei32)
    ps = ps_p.reshape(n_nodes, HID)
    pr = pr_p.reshape(n_nodes, HID)

    slices = ((0, cut, 6400, s_a, r_a), (cut, n_edges, 6400, s_b, r_b))
    pieces = []
    for start, end, rows, s_sl, r_sl in slices:
        e_s = end - start
        half_s = e_s // 2
        g2 = _make_gather(e_s, 200)(ps, pr, s_sl, r_sl)

        nblocks = half_s // rows
        lo_base = start // rows
        hi_base = (start + half_s) // rows
        out_lo, out_hi = pl.pallas_call(
            _mlp_body,
            grid=(nblocks,),
            in_specs=[
                pl.BlockSpec((rows // 2, 2 * HID), lambda i: (2 * i, 0)),
                pl.BlockSpec((rows // 2, 2 * HID), lambda i: (2 * i + 1, 0)),
                pl.BlockSpec((EDGE_D, rows), lambda i, _b=lo_base: (0, i + _b)),
                pl.BlockSpec((EDGE_D, rows), lambda i, _b=hi_base: (0, i + _b)),
                pl.BlockSpec((EDGE_D, HID), lambda i: (0, 0)),
                pl.BlockSpec((1, HID), lambda i: (0, 0)),
                pl.BlockSpec((HID, HID), lambda i: (0, 0)),
                pl.BlockSpec((1, HID), lambda i: (0, 0)),
                pl.BlockSpec((1, HID), lambda i: (0, 0)),
                pl.BlockSpec((1, 1), lambda i: (0, 0)),
            ],
            out_specs=(
                pl.BlockSpec((1, 1, rows), lambda i: (i, 0, 0)),
                pl.BlockSpec((1, 1, rows), lambda i: (i, 0, 0)),
            ),
            out_shape=(
                jax.ShapeDtypeStruct((nblocks, 1, rows), jnp.float32),
                jax.ShapeDtypeStruct((nblocks, 1, rows), jnp.float32),
            ),
        )(g2, g2, eft, eft, w1et, b1_2, W2.T, b2_2, W3, b3_2)
        pieces.append(out_lo.reshape(half_s))
        pieces.append(out_hi.reshape(half_s))
    return jnp.concatenate(pieces)


# speedup vs baseline: 1.9337x; 1.1659x over previous
import functools

import jax
import jax.numpy as jnp
from jax import lax
from jax.experimental import pallas as pl
from jax.experimental.pallas import tpu as pltpu
from jax.experimental.pallas import tpu_sc as plsc

NODE_D = 128
HID = 64
EDGE_D = 16

_NC = 2
_NS = 16
_NW = _NC * _NS


def _elu(x):
    return jnp.where(x > 0, x, jnp.exp(x) - 1.0)


def _softplus(x):
    return jnp.maximum(x, 0.0) + jnp.log(1.0 + jnp.exp(-jnp.abs(x)))


def _precompute_body(nf_ref, wst_ref, wrt_ref, ei_ref, ps_ref, pr_ref,
                     sa_ref, ra_ref):
    nf = nf_ref[...]
    hn = nf.shape[0] // 2
    wst = wst_ref[...]
    wrt = wrt_ref[...]
    ps_ref[...] = jnp.concatenate(
        [jnp.dot(nf[:hn], wst, preferred_element_type=jnp.float32),
         jnp.dot(nf[hn:], wst, preferred_element_type=jnp.float32)], axis=1)
    pr_ref[...] = jnp.concatenate(
        [jnp.dot(nf[:hn], wrt, preferred_element_type=jnp.float32),
         jnp.dot(nf[hn:], wrt, preferred_element_type=jnp.float32)], axis=1)
    s_row = ei_ref[0]
    r_row = ei_ref[1]
    s_row = 2 * s_row - jnp.where(s_row < hn, 0, 2 * hn - 1)
    r_row = 2 * r_row - jnp.where(r_row < hn, 0, 2 * hn - 1)
    off = 0
    for s_ref, r_ref in zip(sa_ref, ra_ref):
        n = s_ref.shape[0]
        s_ref[...] = s_row[off:off + n]
        r_ref[...] = r_row[off:off + n]
        off += n


def _mlp_body(g2a_ref, g2b_ref, eftlo_ref, efthi_ref, w1ea_ref, w2bd_ref,
              b2p_ref, w3c_ref, b3_ref, outlo_ref, outhi_ref):
    g2 = jnp.concatenate([g2a_ref[...], g2b_ref[...]], axis=0)
    rows = g2.shape[0]
    eft2 = jnp.concatenate(
        [eftlo_ref[...], efthi_ref[...],
         jnp.ones((1, rows), jnp.float32)], axis=0)
    a = lax.dot_general(eft2, w1ea_ref[...], (((0,), (0,)), ((), ())),
                        preferred_element_type=jnp.float32)
    h1 = _elu(g2 + a)
    h2 = _elu(jnp.dot(h1, w2bd_ref[...], preferred_element_type=jnp.float32)
              + b2p_ref[...])
    z = lax.dot_general(w3c_ref[...], h2, (((1,), (1,)), ((), ())),
                        preferred_element_type=jnp.float32)
    sp = _softplus(z + b3_ref[...])
    outlo_ref[0] = sp[0:1]
    outhi_ref[0] = sp[1:2]


def _make_gather(n_edges, cpairs):
    half = n_edges // 2
    ppw = half // _NW
    nchunk = ppw // cpairs
    mesh = plsc.VectorSubcoreMesh(core_axis_name="c", subcore_axis_name="s")

    @functools.partial(
        pl.kernel,
        mesh=mesh,
        compiler_params=pltpu.CompilerParams(use_tc_tiling_on_sc=False),
        out_type=jax.ShapeDtypeStruct((half, 2 * HID), jnp.float32),
        scratch_types=[
            pltpu.VMEM((2 * ppw,), jnp.int32),
            pltpu.VMEM((2 * ppw,), jnp.int32),
            pltpu.VMEM((cpairs, HID), jnp.float32),
            pltpu.VMEM((cpairs, HID), jnp.float32),
            pltpu.VMEM((cpairs, HID), jnp.float32),
            pltpu.VMEM((cpairs, HID), jnp.float32),
            pltpu.VMEM((cpairs, HID), jnp.float32),
            pltpu.VMEM((cpairs, HID), jnp.float32),
            pltpu.SemaphoreType.DMA,
            pltpu.SemaphoreType.DMA,
            pltpu.SemaphoreType.DMA,
            pltpu.SemaphoreType.DMA,
        ],
    )
    def _gather(ps_hbm, pr_hbm, s_hbm, r_hbm, out_hbm, idx_s, idx_r,
                bs0, br0, bw0, bs1, br1, bw1, semg0, semg1, semw0, semw1):
        wid = lax.axis_index("s") * _NC + lax.axis_index("c")
        base = wid * ppw
        bufs = ((bs0, br0, bw0, semg0, semw0), (bs1, br1, bw1, semg1, semw1))

        pltpu.sync_copy(s_hbm.at[pl.ds(base, ppw)], idx_s.at[pl.ds(0, ppw)])
        pltpu.sync_copy(s_hbm.at[pl.ds(half + base, ppw)],
                        idx_s.at[pl.ds(ppw, ppw)])
        pltpu.sync_copy(r_hbm.at[pl.ds(base, ppw)], idx_r.at[pl.ds(0, ppw)])
        pltpu.sync_copy(r_hbm.at[pl.ds(half + base, ppw)],
                        idx_r.at[pl.ds(ppw, ppw)])

        def issue_gathers(b, k):
            bs, br, _, semg, _ = bufs[b]
            ioff = b * ppw + k * cpairs
            cs = pltpu.async_copy(
                ps_hbm.at[idx_s.at[pl.ds(ioff, cpairs)]], bs, semg)
            cr = pltpu.async_copy(
                pr_hbm.at[idx_r.at[pl.ds(ioff, cpairs)]], br, semg)
            return cs, cr

        issue_gathers(0, 0)
        issue_gathers(1, 0)

        def chunk_body(k, carry):
            for b in (0, 1):
                bs, br, bw, semg, semw = bufs[b]
                coff = b * HID
                row = base + k * cpairs

                pltpu.make_async_copy(
                    ps_hbm.at[idx_s.at[pl.ds(0, cpairs)]], bs, semg).wait()
                pltpu.make_async_copy(
                    pr_hbm.at[idx_r.at[pl.ds(0, cpairs)]], br, semg).wait()

                @pl.when(k >= 1)
                def _():
                    pltpu.make_async_copy(
                        bw, out_hbm.at[pl.ds(row - cpairs, cpairs),
                                       pl.ds(coff, HID)], semw).wait()

                def add_row(rr, inner):
                    for c4 in range(HID // 16):
                        sl = pl.ds(c4 * 16, 16)
                        bw[rr, sl] = bs[rr, sl] + br[rr, sl]
                    return inner

                lax.fori_loop(0, cpairs, add_row, 0)

                @pl.when(k < nchunk - 1)
                def _():
                    issue_gathers(b, k + 1)

                pltpu.async_copy(
                    bw, out_hbm.at[pl.ds(row, cpairs), pl.ds(coff, HID)], semw)
            return carry

        lax.fori_loop(0, nchunk, chunk_body, 0)

        for b in (0, 1):
            _, _, bw, _, semw = bufs[b]
            row = base + (nchunk - 1) * cpairs
            pltpu.make_async_copy(
                bw, out_hbm.at[pl.ds(row, cpairs), pl.ds(b * HID, HID)],
                semw).wait()

    return _gather


def kernel(node_features, edge_index, edge_features, W1, b1, W2, b2, W3, b3):
    n_nodes = node_features.shape[0]
    n_edges = edge_features.shape[0]
    half = n_edges // 2

    eft = edge_features.T
    w1st = W1[:, :NODE_D].T
    w1rt = W1[:, NODE_D:2 * NODE_D].T
    w1et = W1[:, 2 * NODE_D:].T
    b1_2 = b1.reshape(1, HID)
    b2_2 = b2.reshape(1, HID)
    b3_2 = b3.reshape(1, 1)

    sizes = (102400, 102400, 115200)
    rows = 3200
    ei32 = jnp.asarray(edge_index, jnp.int32)

    zer_e = jnp.zeros((EDGE_D, HID), jnp.float32)
    w1ea = jnp.concatenate([
        jnp.concatenate([w1et, zer_e], axis=1),
        jnp.concatenate([zer_e, w1et], axis=1),
        jnp.concatenate([b1_2, b1_2], axis=1),
    ], axis=0)
    zer_h = jnp.zeros((HID, HID), jnp.float32)
    w2bd = jnp.block([[W2.T, zer_h], [zer_h, W2.T]])
    b2p = jnp.concatenate([b2_2, b2_2], axis=1)
    zer_3 = jnp.zeros((1, HID), jnp.float32)
    w3c = jnp.block([[W3, zer_3], [zer_3, W3]])

    ps_p, pr_p, s_sl, r_sl = pl.pallas_call(
        _precompute_body,
        out_shape=(
            jax.ShapeDtypeStruct((n_nodes // 2, 2 * HID), jnp.float32),
            jax.ShapeDtypeStruct((n_nodes // 2, 2 * HID), jnp.float32),
            tuple(jax.ShapeDtypeStruct((n,), jnp.int32) for n in sizes),
            tuple(jax.ShapeDtypeStruct((n,), jnp.int32) for n in sizes),
        ),
    )(node_features, w1st, w1rt, ei32)
    ps = ps_p.reshape(n_nodes, HID)
    pr = pr_p.reshape(n_nodes, HID)

    pieces = []
    start = 0
    for e_s, s_k, r_k in zip(sizes, s_sl, r_sl):
        half_s = e_s // 2
        g2 = _make_gather(e_s, 200)(ps, pr, s_k, r_k)

        nblocks = half_s // rows
        lo_base = start // rows
        hi_base = (start + half_s) // rows
        out_lo, out_hi = pl.pallas_call(
            _mlp_body,
            grid=(nblocks,),
            in_specs=[
                pl.BlockSpec((rows // 2, 2 * HID), lambda i: (2 * i, 0)),
                pl.BlockSpec((rows // 2, 2 * HID), lambda i: (2 * i + 1, 0)),
                pl.BlockSpec((EDGE_D, rows), lambda i, _b=lo_base: (0, i + _b)),
                pl.BlockSpec((EDGE_D, rows), lambda i, _b=hi_base: (0, i + _b)),
                pl.BlockSpec((2 * EDGE_D + 1, 2 * HID), lambda i: (0, 0)),
                pl.BlockSpec((2 * HID, 2 * HID), lambda i: (0, 0)),
                pl.BlockSpec((1, 2 * HID), lambda i: (0, 0)),
                pl.BlockSpec((2, 2 * HID), lambda i: (0, 0)),
                pl.BlockSpec((1, 1), lambda i: (0, 0)),
            ],
            out_specs=(
                pl.BlockSpec((1, 1, rows), lambda i: (i, 0, 0)),
                pl.BlockSpec((1, 1, rows), lambda i: (i, 0, 0)),
            ),
            out_shape=(
                jax.ShapeDtypeStruct((nblocks, 1, rows), jnp.float32),
                jax.ShapeDtypeStruct((nblocks, 1, rows), jnp.float32),
            ),
        )(g2, g2, eft, eft, w1ea, w2bd, b2p, w3c, b3_2)
        pieces.append(out_lo.reshape(half_s))
        pieces.append(out_hi.reshape(half_s))
        start += e_s
    return jnp.concatenate(pieces)
